# baseline (speedup 1.0000x reference)
import functools

import numpy as np
import jax
import jax.numpy as jnp
from jax import lax
from jax.experimental import pallas as pl
from jax.experimental.pallas import tpu as pltpu
from jax.experimental.pallas import tpu_sc as plsc

_N = 8192
_m = 4
_n = 16384
_B = 16
_M = 65536
_b = (2.0 - 1.0 / 2.0) * np.pi

_NTAB = _n + 8
_HALF = _M // 2
_CHUNK = 16384
_DEG = 4


def _tap_coeffs():
    u = np.linspace(-1.0, 0.0, 8001)
    coefs = []
    for l in range(8):
        t = u + 4.0 - l
        v = 16.0 - t * t
        s = np.sqrt(np.maximum(v, 0.0))
        y = np.where(v > 1e-30, np.sinh(_b * s) / (np.pi * np.maximum(s, 1e-300)), _b / np.pi)
        c = np.polynomial.chebyshev.Chebyshev.fit(u, y, _DEG, domain=[-1.0, 0.0])
        p = c.convert(kind=np.polynomial.Polynomial)
        coefs.append([float(cc) for cc in p.coef])
    return coefs


_COEF = _tap_coeffs()


def _sc_body(x_hbm, gre_hbm, gim_hbm, fre_hbm, fim_hbm, gre_v, gim_v, x_v, ore_v, oim_v):
    wid = lax.axis_index("s") * 2 + lax.axis_index("c")
    batch = wid // 2
    half = wid % 2
    pltpu.sync_copy(gre_hbm.at[batch], gre_v)
    pltpu.sync_copy(gim_hbm.at[batch], gim_v)

    for ci in range(_HALF // _CHUNK):
        roff = half * _HALF + ci * _CHUNK
        pltpu.sync_copy(x_hbm.at[batch, pl.ds(roff, _CHUNK)], x_v)

        def body(i, carry):
          for gg in range(2):
            base = i * 32 + gg * 16
            xx = x_v[pl.ds(base, 16)]
            y = xx * np.float32(_n)
            ti = y.astype(jnp.int32)
            tf = ti.astype(jnp.float32)
            up = (y > tf)
            ceil_i = ti + up.astype(jnp.int32)
            u = y - ceil_i.astype(jnp.float32)
            s = (ceil_i + np.int32(_n - _m)) & np.int32(_n - 1)
            accr = None
            acci = None
            for l in range(8):
                cl = _COEF[l]
                w = jnp.full((16,), np.float32(cl[_DEG]), jnp.float32)
                for k in range(_DEG - 1, -1, -1):
                    w = w * u + np.float32(cl[k])
                idx = s + np.int32(l)
                gr = plsc.load_gather(gre_v, [idx])
                gi = plsc.load_gather(gim_v, [idx])
                if accr is None:
                    accr, acci = w * gr, w * gi
                else:
                    accr = accr + w * gr
                    acci = acci + w * gi
            ore_v[pl.ds(base, 16)] = accr
            oim_v[pl.ds(base, 16)] = acci
          return carry

        lax.fori_loop(0, _CHUNK // 32, body, 0)

        pltpu.sync_copy(ore_v, fre_hbm.at[batch, pl.ds(roff, _CHUNK)])
        pltpu.sync_copy(oim_v, fim_hbm.at[batch, pl.ds(roff, _CHUNK)])


@functools.lru_cache(maxsize=1)
def _build_sc_conv():
    mesh = plsc.VectorSubcoreMesh(core_axis_name="c", subcore_axis_name="s",
                                  num_cores=2, num_subcores=16)
    return pl.kernel(
        _sc_body,
        out_type=(jax.ShapeDtypeStruct((_B, _M), jnp.float32),
                  jax.ShapeDtypeStruct((_B, _M), jnp.float32)),
        mesh=mesh,
        compiler_params=pltpu.CompilerParams(needs_layout_passes=False),
        scratch_types=[
            pltpu.VMEM((_NTAB,), jnp.float32),
            pltpu.VMEM((_NTAB,), jnp.float32),
            pltpu.VMEM((_CHUNK,), jnp.float32),
            pltpu.VMEM((_CHUNK,), jnp.float32),
            pltpu.VMEM((_CHUNK,), jnp.float32),
        ],
    )


def _phi_hat():
    inds = jnp.arange(-_N // 2, _N // 2, dtype=jnp.float32)
    return jax.scipy.special.i0(_m * jnp.sqrt(_b ** 2 - (2.0 * jnp.pi * inds / _n) ** 2))


def kernel(x, f_hat):
    g_hat = f_hat / _phi_hat()
    zeros = jnp.zeros((x.shape[0], _n - _N), dtype=g_hat.dtype)
    h1 = jnp.concatenate((g_hat[:, _N // 2:], zeros, g_hat[:, : _N // 2]), axis=1)
    F = jnp.fft.fft(h1)
    tab = jnp.concatenate((F, F[:, :8]), axis=1)
    gre = jnp.real(tab).astype(jnp.float32)
    gim = jnp.imag(tab).astype(jnp.float32)
    fre, fim = _build_sc_conv()(x, gre, gim)
    return lax.complex(fre, fim)


# speedup vs baseline: 1.0465x; 1.0465x over previous
import functools

import numpy as np
import jax
import jax.numpy as jnp
from jax import lax
from jax.experimental import pallas as pl
from jax.experimental.pallas import tpu as pltpu
from jax.experimental.pallas import tpu_sc as plsc

_N = 8192
_m = 4
_n = 16384
_B = 16
_M = 65536
_b = (2.0 - 1.0 / 2.0) * np.pi

_NTAB = _n + 8
_HALF = _M // 2
_CHUNK = 8192
_DEG = 4


def _tap_coeffs():
    u = np.linspace(-1.0, 0.0, 8001)
    coefs = []
    for l in range(8):
        t = u + 4.0 - l
        v = 16.0 - t * t
        s = np.sqrt(np.maximum(v, 0.0))
        y = np.where(v > 1e-30, np.sinh(_b * s) / (np.pi * np.maximum(s, 1e-300)), _b / np.pi)
        c = np.polynomial.chebyshev.Chebyshev.fit(u, y, _DEG, domain=[-1.0, 0.0])
        p = c.convert(kind=np.polynomial.Polynomial)
        coefs.append([float(cc) for cc in p.coef])
    return coefs


_COEF = _tap_coeffs()


def _sc_body(x_hbm, gre_hbm, gim_hbm, fre_hbm, fim_hbm,
             gre_v, gim_v, x_v0, x_v1, ore_v0, ore_v1, oim_v0, oim_v1,
             sem_t0, sem_t1, sem_x0, sem_x1, sem_o0, sem_o1):
    wid = lax.axis_index("s") * 2 + lax.axis_index("c")
    batch = wid // 2
    half = wid % 2
    xbuf = (x_v0, x_v1)
    rebuf = (ore_v0, ore_v1)
    imbuf = (oim_v0, oim_v1)
    sem_x = (sem_x0, sem_x1)
    sem_o = (sem_o0, sem_o1)
    nchunks = _HALF // _CHUNK

    def x_copy(ci):
        roff = half * _HALF + ci * _CHUNK
        return pltpu.async_copy(x_hbm.at[batch, pl.ds(roff, _CHUNK)],
                                xbuf[ci % 2], sem_x[ci % 2])

    t0 = pltpu.async_copy(gre_hbm.at[batch], gre_v, sem_t0)
    t1 = pltpu.async_copy(gim_hbm.at[batch], gim_v, sem_t1)
    xd = x_copy(0)
    t0.wait()
    t1.wait()
    out_pending = [None, None]

    for ci in range(nchunks):
        cur = ci % 2
        roff = half * _HALF + ci * _CHUNK
        xd.wait()
        if ci + 1 < nchunks:
            xd = x_copy(ci + 1)
        if out_pending[cur] is not None:
            for d in out_pending[cur]:
                d.wait()
        x_v = xbuf[cur]
        ore_v = rebuf[cur]
        oim_v = imbuf[cur]

        def body(i, carry):
            base = i * 16
            xx = x_v[pl.ds(base, 16)]
            y = xx * np.float32(_n)
            ti = y.astype(jnp.int32)
            tf = ti.astype(jnp.float32)
            up = (y > tf)
            ceil_i = ti + up.astype(jnp.int32)
            u = y - ceil_i.astype(jnp.float32)
            s = (ceil_i + np.int32(_n - _m)) & np.int32(_n - 1)
            accr = None
            acci = None
            for l in range(8):
                cl = _COEF[l]
                w = jnp.full((16,), np.float32(cl[_DEG]), jnp.float32)
                for k in range(_DEG - 1, -1, -1):
                    w = w * u + np.float32(cl[k])
                idx = s + np.int32(l)
                gr = plsc.load_gather(gre_v, [idx])
                gi = plsc.load_gather(gim_v, [idx])
                if accr is None:
                    accr, acci = w * gr, w * gi
                else:
                    accr = accr + w * gr
                    acci = acci + w * gi
            ore_v[pl.ds(base, 16)] = accr
            oim_v[pl.ds(base, 16)] = acci
            return carry

        lax.fori_loop(0, _CHUNK // 16, body, 0)

        out_pending[cur] = (
            pltpu.async_copy(ore_v, fre_hbm.at[batch, pl.ds(roff, _CHUNK)], sem_o[cur]),
            pltpu.async_copy(oim_v, fim_hbm.at[batch, pl.ds(roff, _CHUNK)], sem_o[cur]),
        )

    for pend in out_pending:
        if pend is not None:
            for d in pend:
                d.wait()


@functools.lru_cache(maxsize=1)
def _build_sc_conv():
    mesh = plsc.VectorSubcoreMesh(core_axis_name="c", subcore_axis_name="s",
                                  num_cores=2, num_subcores=16)
    return pl.kernel(
        _sc_body,
        out_type=(jax.ShapeDtypeStruct((_B, _M), jnp.float32),
                  jax.ShapeDtypeStruct((_B, _M), jnp.float32)),
        mesh=mesh,
        compiler_params=pltpu.CompilerParams(needs_layout_passes=False),
        scratch_types=[
            pltpu.VMEM((_NTAB,), jnp.float32),
            pltpu.VMEM((_NTAB,), jnp.float32),
            pltpu.VMEM((_CHUNK,), jnp.float32),
            pltpu.VMEM((_CHUNK,), jnp.float32),
            pltpu.VMEM((_CHUNK,), jnp.float32),
            pltpu.VMEM((_CHUNK,), jnp.float32),
            pltpu.VMEM((_CHUNK,), jnp.float32),
            pltpu.VMEM((_CHUNK,), jnp.float32),
            pltpu.SemaphoreType.DMA,
            pltpu.SemaphoreType.DMA,
            pltpu.SemaphoreType.DMA,
            pltpu.SemaphoreType.DMA,
            pltpu.SemaphoreType.DMA,
            pltpu.SemaphoreType.DMA,
        ],
    )


def _phi_hat():
    inds = jnp.arange(-_N // 2, _N // 2, dtype=jnp.float32)
    return jax.scipy.special.i0(_m * jnp.sqrt(_b ** 2 - (2.0 * jnp.pi * inds / _n) ** 2))


def kernel(x, f_hat):
    g_hat = f_hat / _phi_hat()
    zeros = jnp.zeros((x.shape[0], _n - _N), dtype=g_hat.dtype)
    h1 = jnp.concatenate((g_hat[:, _N // 2:], zeros, g_hat[:, : _N // 2]), axis=1)
    F = jnp.fft.fft(h1)
    tab = jnp.concatenate((F, F[:, :8]), axis=1)
    gre = jnp.real(tab).astype(jnp.float32)
    gim = jnp.imag(tab).astype(jnp.float32)
    fre, fim = _build_sc_conv()(x, gre, gim)
    return lax.complex(fre, fim)
